# Initial kernel scaffold; baseline (speedup 1.0000x reference)
import jax
import jax.numpy as jnp
from jax.experimental import pallas as pl


def kernel(x, edge_index, emb, W1, b1, W2, b2, Wfc, bfc):
    raise NotImplementedError("write your pallas kernel here")


# speedup vs baseline: 11.4074x; 11.4074x over previous
import dataclasses
import functools

import jax
import jax.numpy as jnp
from jax import lax
from jax.experimental import pallas as pl
from jax.experimental.pallas import tpu as pltpu
from jax.experimental.pallas import tpu_sc as plsc

N = 50000
E = 800000
H = 64
HH = 32
NTAGS = 1000
NC = 2
NS = 16
L = 16

NPH = 51200
HSLICE = NPH // NS
EPT_H = E // (NC * NS)
NPG = 50176
GPT = NPG // (NC * NS)
EPT = E // NS
ROWS_PT = N // NS
ZROWS = 125

def _sc_params():
    cp = pltpu.CompilerParams()
    fields = pltpu.CompilerParams.__dataclass_fields__
    if "needs_layout_passes" in fields:
        cp = dataclasses.replace(cp, needs_layout_passes=False)
    if "use_tc_tiling_on_sc" in fields:
        cp = dataclasses.replace(cp, use_tc_tiling_on_sc=False)
    return cp


@functools.lru_cache(maxsize=1)
def _mesh():
    return plsc.VectorSubcoreMesh(core_axis_name="c", subcore_axis_name="s",
                                  num_cores=NC, num_subcores=NS)
_f32 = jnp.float32


def _zero16():
    return jnp.zeros((L,), _f32)


@jax.jit
def _sc_hist(col):
    @functools.partial(
        pl.kernel,
        out_type=jax.ShapeDtypeStruct((NC, NPH), _f32),
        mesh=_mesh(),
        scratch_types=[
            pltpu.VMEM((1600,), jnp.int32),
            pltpu.VMEM((NPH,), _f32),
            pltpu.VMEM((HSLICE,), _f32),
            pltpu.VMEM((HSLICE,), _f32),
            pltpu.VMEM_SHARED((NS, NPH), _f32),
        ],
        compiler_params=_sc_params(),
    )
    def k(col_hbm, out_hbm, colbuf, hist, accv, tmpv, stage):
        c = lax.axis_index("c")
        s = lax.axis_index("s")
        ones = jnp.ones((L,), _f32)

        @pl.loop(0, NPH, step=L)
        def _(i):
            hist[pl.ds(i, L)] = _zero16()

        e0 = (c * NS + s) * EPT_H

        @pl.loop(0, 15)
        def _(j):
            pltpu.sync_copy(col_hbm.at[pl.ds(e0 + j * 1600, 1600)], colbuf)

            @pl.loop(0, 1600, step=L)
            def _(i):
                plsc.addupdate_scatter(hist, [colbuf[pl.ds(i, L)]], ones)

        pltpu.sync_copy(col_hbm.at[pl.ds(e0 + 24000, 1000)],
                        colbuf.at[pl.ds(0, 1000)])

        @pl.loop(0, 992, step=L)
        def _(i):
            plsc.addupdate_scatter(hist, [colbuf[pl.ds(i, L)]], ones)

        tail_mask = lax.iota(jnp.int32, L) < 8
        plsc.addupdate_scatter(hist, [colbuf[pl.ds(992, L)]], ones,
                               mask=tail_mask)

        pltpu.sync_copy(hist, stage.at[s])
        plsc.subcore_barrier()
        base = s * HSLICE

        @pl.loop(0, HSLICE, step=L)
        def _(i):
            accv[pl.ds(i, L)] = _zero16()

        @pl.loop(0, NS)
        def _(t):
            pltpu.sync_copy(stage.at[t, pl.ds(base, HSLICE)], tmpv)

            @pl.loop(0, HSLICE, step=L)
            def _(i):
                accv[pl.ds(i, L)] = accv[pl.ds(i, L)] + tmpv[pl.ds(i, L)]

        pltpu.sync_copy(accv, out_hbm.at[c, pl.ds(base, HSLICE)])

    return k(col)


@jax.jit
def _sc_gather(table, idx):
    @functools.partial(
        pl.kernel,
        out_type=jax.ShapeDtypeStruct((NPG, H), _f32),
        mesh=_mesh(),
        scratch_types=[
            pltpu.VMEM((128,), jnp.int32),
            pltpu.VMEM((128, H), _f32),
            pltpu.VMEM((32,), jnp.int32),
            pltpu.VMEM((32, H), _f32),
            pltpu.SemaphoreType.DMA,
        ],
        compiler_params=_sc_params(),
    )
    def k(t_hbm, x_hbm, out_hbm, idxv, rows, idxt, rowst, sem):
        w = lax.axis_index("s") * NC + lax.axis_index("c")
        base = w * GPT

        @pl.loop(0, 12)
        def _(j):
            b = base + j * 128
            pltpu.sync_copy(x_hbm.at[pl.ds(b, 128)], idxv)
            pltpu.async_copy(t_hbm.at[idxv], rows, sem).wait()
            pltpu.sync_copy(rows, out_hbm.at[pl.ds(b, 128), :])

        b = base + 1536
        pltpu.sync_copy(x_hbm.at[pl.ds(b, 32)], idxt)
        pltpu.async_copy(t_hbm.at[idxt], rowst, sem).wait()
        pltpu.sync_copy(rowst, out_hbm.at[pl.ds(b, 32), :])

    return k(table, idx)


@jax.jit
def _sc_edge(g2d, row, col):
    @functools.partial(
        pl.kernel,
        out_type=jax.ShapeDtypeStruct((NC * N, HH), _f32),
        mesh=_mesh(),
        scratch_types=[
            pltpu.VMEM((128,), jnp.int32),
            pltpu.VMEM((128,), jnp.int32),
            pltpu.VMEM((128, HH), _f32),
            pltpu.VMEM((80,), jnp.int32),
            pltpu.VMEM((80,), jnp.int32),
            pltpu.VMEM((80, HH), _f32),
            pltpu.VMEM((ZROWS, HH), _f32),
            pltpu.VMEM_SHARED((N, HH), _f32),
            pltpu.SemaphoreType.DMA,
        ],
        compiler_params=_sc_params(),
    )
    def k(g_hbm, row_hbm, col_hbm, out_hbm,
          ir, ic, rows, irt, ict, rowst, zb, acc, sem):
        c = lax.axis_index("c")
        s = lax.axis_index("s")

        @pl.loop(0, ZROWS)
        def _(i):
            zb[i, pl.ds(0, L)] = _zero16()
            zb[i, pl.ds(L, L)] = _zero16()

        r0 = s * ROWS_PT

        @pl.loop(0, ROWS_PT // ZROWS)
        def _(i):
            pltpu.sync_copy(zb, acc.at[pl.ds(r0 + i * ZROWS, ZROWS), :])

        plsc.subcore_barrier()

        off = c * N
        e0 = s * EPT

        @pl.loop(0, EPT // 128)
        def _(j):
            b = e0 + j * 128
            pltpu.sync_copy(row_hbm.at[pl.ds(b, 128)], ir)
            pltpu.sync_copy(col_hbm.at[pl.ds(b, 128)], ic)

            @pl.loop(0, 128, step=L)
            def _(i):
                ir[pl.ds(i, L)] = ir[pl.ds(i, L)] + off

            pltpu.async_copy(g_hbm.at[ir], rows, sem).wait()
            pltpu.sync_copy(rows, acc.at[ic], add=True)

        b = e0 + (EPT // 128) * 128
        pltpu.sync_copy(row_hbm.at[pl.ds(b, 80)], irt)
        pltpu.sync_copy(col_hbm.at[pl.ds(b, 80)], ict)

        @pl.loop(0, 80, step=L)
        def _(i):
            irt[pl.ds(i, L)] = irt[pl.ds(i, L)] + off

        pltpu.async_copy(g_hbm.at[irt], rowst, sem).wait()
        pltpu.sync_copy(rowst, acc.at[ict], add=True)

        plsc.subcore_barrier()

        @pl.loop(0, ROWS_PT // ZROWS)
        def _(i):
            rr = r0 + i * ZROWS
            pltpu.sync_copy(acc.at[pl.ds(rr, ZROWS), :],
                            out_hbm.at[pl.ds(off + rr, ZROWS), :])

    return k(g2d, row, col)


R = 400


def _mm_small(a, b):
    def body(a_ref, b_ref, o_ref):
        o_ref[...] = jnp.dot(a_ref[...], b_ref[...],
                             preferred_element_type=_f32)

    return pl.pallas_call(
        body,
        out_shape=jax.ShapeDtypeStruct((a.shape[0], b.shape[1]), _f32),
    )(a, b)


def _tc_scale(h0, h1, lin1p):
    def body(h0_ref, h1_ref, lin_ref, g_ref, dis_ref):
        deg = h0_ref[...] + h1_ref[...] + 1.0
        dis = lax.rsqrt(deg)
        g = lin_ref[...] * dis
        g_ref[...] = jnp.stack([g[:, :HH], g[:, HH:]], axis=0)
        dis_ref[...] = dis

    return pl.pallas_call(
        body,
        grid=(N // R,),
        in_specs=[
            pl.BlockSpec((R, 1), lambda i: (i, 0)),
            pl.BlockSpec((R, 1), lambda i: (i, 0)),
            pl.BlockSpec((R, H), lambda i: (i, 0)),
        ],
        out_specs=[
            pl.BlockSpec((NC, R, HH), lambda i: (0, i, 0)),
            pl.BlockSpec((R, 1), lambda i: (i, 0)),
        ],
        out_shape=[
            jax.ShapeDtypeStruct((NC, N, HH), _f32),
            jax.ShapeDtypeStruct((N, 1), _f32),
        ],
    )(h0, h1, lin1p)


def _tc_combine(acc2d, g2d, dis, b, w2):
    def body(al_ref, ah_ref, gl_ref, gh_ref, dis_ref, b_ref, w_ref, o_ref):
        a64 = jnp.concatenate([al_ref[...], ah_ref[...]], axis=1)
        g64 = jnp.concatenate([gl_ref[...], gh_ref[...]], axis=1)
        d = dis_ref[...]
        h = jnp.maximum(d * (a64 + g64) + b_ref[...], 0.0)
        lin = jnp.dot(h, w_ref[...], preferred_element_type=_f32)
        gn = lin * d
        o_ref[...] = jnp.stack([gn[:, :HH], gn[:, HH:]], axis=0)

    return pl.pallas_call(
        body,
        grid=(N // R,),
        in_specs=[
            pl.BlockSpec((R, HH), lambda i: (i, 0)),
            pl.BlockSpec((R, HH), lambda i: (i + N // R, 0)),
            pl.BlockSpec((R, HH), lambda i: (i, 0)),
            pl.BlockSpec((R, HH), lambda i: (i + N // R, 0)),
            pl.BlockSpec((R, 1), lambda i: (i, 0)),
            pl.BlockSpec((1, H), lambda i: (0, 0)),
            pl.BlockSpec((H, H), lambda i: (0, 0)),
        ],
        out_specs=pl.BlockSpec((NC, R, HH), lambda i: (0, i, 0)),
        out_shape=jax.ShapeDtypeStruct((NC, N, HH), _f32),
    )(acc2d, acc2d, g2d, g2d, dis, b, w2)


def _tc_final(acc2d, g2d, dis, b, wfc, bfc):
    def body(al_ref, ah_ref, gl_ref, gh_ref, dis_ref, b_ref, w_ref,
             bfc_ref, o_ref):
        a64 = jnp.concatenate([al_ref[...], ah_ref[...]], axis=1)
        g64 = jnp.concatenate([gl_ref[...], gh_ref[...]], axis=1)
        d = dis_ref[...]
        h = jnp.maximum(d * (a64 + g64) + b_ref[...], 0.0)
        o_ref[...] = (jnp.dot(h, w_ref[...], preferred_element_type=_f32)
                      + bfc_ref[0, 0])

    return pl.pallas_call(
        body,
        grid=(N // R,),
        in_specs=[
            pl.BlockSpec((R, HH), lambda i: (i, 0)),
            pl.BlockSpec((R, HH), lambda i: (i + N // R, 0)),
            pl.BlockSpec((R, HH), lambda i: (i, 0)),
            pl.BlockSpec((R, HH), lambda i: (i + N // R, 0)),
            pl.BlockSpec((R, 1), lambda i: (i, 0)),
            pl.BlockSpec((1, H), lambda i: (0, 0)),
            pl.BlockSpec((H, 1), lambda i: (0, 0)),
            pl.BlockSpec((1, 1), lambda i: (0, 0)),
        ],
        out_specs=pl.BlockSpec((R, 1), lambda i: (i, 0)),
        out_shape=jax.ShapeDtypeStruct((N, 1), _f32),
    )(acc2d, acc2d, g2d, g2d, dis, b, wfc, bfc)


@jax.jit
def kernel(x, edge_index, emb, W1, b1, W2, b2, Wfc, bfc):
    row = edge_index[0]
    col = edge_index[1]

    t1 = _mm_small(emb, W1)
    hist2 = _sc_hist(col)
    xp = jnp.pad(x, (0, NPG - N))
    lin1p = _sc_gather(t1, xp)

    g1_3d, dis = _tc_scale(hist2[0].reshape(NPH, 1), hist2[1].reshape(NPH, 1),
                           lin1p)
    g1 = g1_3d.reshape(NC * N, HH)

    a1 = _sc_edge(g1, row, col)
    g2_3d = _tc_combine(a1, g1, dis, b1.reshape(1, H), W2)
    g2 = g2_3d.reshape(NC * N, HH)

    a2 = _sc_edge(g2, row, col)
    out = _tc_final(a2, g2, dis, b2.reshape(1, H), Wfc, bfc.reshape(1, 1))
    return out.reshape(N)


# speedup vs baseline: 18.2241x; 1.5976x over previous
import dataclasses
import functools

import jax
import jax.numpy as jnp
from jax import lax
from jax.experimental import pallas as pl
from jax.experimental.pallas import tpu as pltpu
from jax.experimental.pallas import tpu_sc as plsc

N = 50000
E = 800000
H = 64
HH = 32
NTAGS = 1000
NC = 2
NS = 16
L = 16

NPH = 51200
HSLICE = NPH // NS
EPT_H = E // (NC * NS)
NPG = 50176
GPT = NPG // (NC * NS)
CH = 128
KB = 5
GRP = KB * CH
EPAD = 808960
EPT = EPAD // NS
NGRP = EPT // GRP
NPAD = 50400
RPT = NPAD // NS
FCH = 150
NF = RPT // FCH
NBLK_HI = NPAD // 400

def _sc_params():
    cp = pltpu.CompilerParams()
    fields = pltpu.CompilerParams.__dataclass_fields__
    if "needs_layout_passes" in fields:
        cp = dataclasses.replace(cp, needs_layout_passes=False)
    if "use_tc_tiling_on_sc" in fields:
        cp = dataclasses.replace(cp, use_tc_tiling_on_sc=False)
    return cp


@functools.lru_cache(maxsize=1)
def _mesh():
    return plsc.VectorSubcoreMesh(core_axis_name="c", subcore_axis_name="s",
                                  num_cores=NC, num_subcores=NS)
_f32 = jnp.float32


def _zero16():
    return jnp.zeros((L,), _f32)


@jax.jit
def _sc_hist(col):
    @functools.partial(
        pl.kernel,
        out_type=jax.ShapeDtypeStruct((NC, NPH), _f32),
        mesh=_mesh(),
        scratch_types=[
            pltpu.VMEM((1600,), jnp.int32),
            pltpu.VMEM((NPH,), _f32),
            pltpu.VMEM((HSLICE,), _f32),
            pltpu.VMEM((HSLICE,), _f32),
            pltpu.VMEM_SHARED((NS, NPH), _f32),
        ],
        compiler_params=_sc_params(),
    )
    def k(col_hbm, out_hbm, colbuf, hist, accv, tmpv, stage):
        c = lax.axis_index("c")
        s = lax.axis_index("s")
        ones = jnp.ones((L,), _f32)

        @pl.loop(0, NPH, step=L)
        def _(i):
            hist[pl.ds(i, L)] = _zero16()

        e0 = (c * NS + s) * EPT_H

        @pl.loop(0, 15)
        def _(j):
            pltpu.sync_copy(col_hbm.at[pl.ds(e0 + j * 1600, 1600)], colbuf)

            @pl.loop(0, 1600, step=L)
            def _(i):
                plsc.addupdate_scatter(hist, [colbuf[pl.ds(i, L)]], ones)

        pltpu.sync_copy(col_hbm.at[pl.ds(e0 + 24000, 1000)],
                        colbuf.at[pl.ds(0, 1000)])

        @pl.loop(0, 992, step=L)
        def _(i):
            plsc.addupdate_scatter(hist, [colbuf[pl.ds(i, L)]], ones)

        tail_mask = lax.iota(jnp.int32, L) < 8
        plsc.addupdate_scatter(hist, [colbuf[pl.ds(992, L)]], ones,
                               mask=tail_mask)

        pltpu.sync_copy(hist, stage.at[s])
        plsc.subcore_barrier()
        base = s * HSLICE

        @pl.loop(0, HSLICE, step=L)
        def _(i):
            accv[pl.ds(i, L)] = _zero16()

        @pl.loop(0, NS)
        def _(t):
            pltpu.sync_copy(stage.at[t, pl.ds(base, HSLICE)], tmpv)

            @pl.loop(0, HSLICE, step=L)
            def _(i):
                accv[pl.ds(i, L)] = accv[pl.ds(i, L)] + tmpv[pl.ds(i, L)]

        pltpu.sync_copy(accv, out_hbm.at[c, pl.ds(base, HSLICE)])

    return k(col)


@jax.jit
def _sc_gather(table, idx):
    @functools.partial(
        pl.kernel,
        out_type=jax.ShapeDtypeStruct((NPG, H), _f32),
        mesh=_mesh(),
        scratch_types=[
            pltpu.VMEM((128,), jnp.int32),
            pltpu.VMEM((128, H), _f32),
            pltpu.VMEM((32,), jnp.int32),
            pltpu.VMEM((32, H), _f32),
            pltpu.SemaphoreType.DMA,
        ],
        compiler_params=_sc_params(),
    )
    def k(t_hbm, x_hbm, out_hbm, idxv, rows, idxt, rowst, sem):
        w = lax.axis_index("s") * NC + lax.axis_index("c")
        base = w * GPT

        @pl.loop(0, 12)
        def _(j):
            b = base + j * 128
            pltpu.sync_copy(x_hbm.at[pl.ds(b, 128)], idxv)
            pltpu.async_copy(t_hbm.at[idxv], rows, sem).wait()
            pltpu.sync_copy(rows, out_hbm.at[pl.ds(b, 128), :])

        b = base + 1536
        pltpu.sync_copy(x_hbm.at[pl.ds(b, 32)], idxt)
        pltpu.async_copy(t_hbm.at[idxt], rowst, sem).wait()
        pltpu.sync_copy(rowst, out_hbm.at[pl.ds(b, 32), :])

    return k(table, idx)


@jax.jit
def _sc_edge(g2d, edge_st):
    @functools.partial(
        pl.kernel,
        out_type=jax.ShapeDtypeStruct((NC * NPAD, HH), _f32),
        mesh=_mesh(),
        scratch_types=[
            pltpu.VMEM((KB, 2, CH), jnp.int32),
            pltpu.VMEM((KB, CH, HH), _f32),
            pltpu.VMEM((FCH, HH), _f32),
            pltpu.VMEM_SHARED((NPAD, HH), _f32),
            pltpu.SemaphoreType.DMA,
            pltpu.SemaphoreType.DMA,
            pltpu.SemaphoreType.DMA,
        ],
        compiler_params=_sc_params(),
    )
    def k(g_hbm, e_hbm, out_hbm, idxb, rows, zb, acc, sem_a, sem_g, sem_s):
        c = lax.axis_index("c")
        s = lax.axis_index("s")

        @pl.loop(0, FCH)
        def _(i):
            zb[i, pl.ds(0, L)] = _zero16()
            zb[i, pl.ds(L, L)] = _zero16()

        r0 = s * RPT
        zd = [pltpu.async_copy(zb, acc.at[pl.ds(r0 + i * FCH, FCH), :],
                               sem_s) for i in range(NF)]
        for d in zd:
            d.wait()
        plsc.subcore_barrier()

        e0 = s * EPT

        @pl.loop(0, NGRP)
        def _(g):
            base = e0 + g * GRP
            da = [pltpu.async_copy(
                      e_hbm.at[c, :, pl.ds(base + nb * CH, CH)],
                      idxb.at[nb], sem_a) for nb in range(KB)]
            for d in da:
                d.wait()
            dg = [pltpu.async_copy(g_hbm.at[idxb.at[nb, 0]], rows.at[nb],
                                   sem_g) for nb in range(KB)]
            for d in dg:
                d.wait()
            dd = [pltpu.async_copy(rows.at[nb], acc.at[idxb.at[nb, 1]],
                                   sem_s, add=True) for nb in range(KB)]
            for d in dd:
                d.wait()

        plsc.subcore_barrier()
        o0 = c * NPAD + r0
        fd = [pltpu.async_copy(acc.at[pl.ds(r0 + i * FCH, FCH), :],
                               out_hbm.at[pl.ds(o0 + i * FCH, FCH), :],
                               sem_s) for i in range(NF)]
        for d in fd:
            d.wait()

    return k(g2d, edge_st)


R = 400


def _mm_small(a, b):
    def body(a_ref, b_ref, o_ref):
        o_ref[...] = jnp.dot(a_ref[...], b_ref[...],
                             preferred_element_type=_f32)

    return pl.pallas_call(
        body,
        out_shape=jax.ShapeDtypeStruct((a.shape[0], b.shape[1]), _f32),
    )(a, b)


def _tc_scale(h0, h1, lin1p):
    def body(h0_ref, h1_ref, lin_ref, g_ref, dis_ref):
        deg = h0_ref[...] + h1_ref[...] + 1.0
        dis = lax.rsqrt(deg)
        g = lin_ref[...] * dis
        g_ref[...] = jnp.stack([g[:, :HH], g[:, HH:]], axis=0)
        dis_ref[...] = dis

    return pl.pallas_call(
        body,
        grid=(N // R,),
        in_specs=[
            pl.BlockSpec((R, 1), lambda i: (i, 0)),
            pl.BlockSpec((R, 1), lambda i: (i, 0)),
            pl.BlockSpec((R, H), lambda i: (i, 0)),
        ],
        out_specs=[
            pl.BlockSpec((NC, R, HH), lambda i: (0, i, 0)),
            pl.BlockSpec((R, 1), lambda i: (i, 0)),
        ],
        out_shape=[
            jax.ShapeDtypeStruct((NC, N, HH), _f32),
            jax.ShapeDtypeStruct((N, 1), _f32),
        ],
    )(h0, h1, lin1p)


def _tc_combine(acc2d, g2d, dis, b, w2):
    def body(al_ref, ah_ref, gl_ref, gh_ref, dis_ref, b_ref, w_ref, o_ref):
        a64 = jnp.concatenate([al_ref[...], ah_ref[...]], axis=1)
        g64 = jnp.concatenate([gl_ref[...], gh_ref[...]], axis=1)
        d = dis_ref[...]
        h = jnp.maximum(d * (a64 + g64) + b_ref[...], 0.0)
        lin = jnp.dot(h, w_ref[...], preferred_element_type=_f32)
        gn = lin * d
        o_ref[...] = jnp.stack([gn[:, :HH], gn[:, HH:]], axis=0)

    return pl.pallas_call(
        body,
        grid=(N // R,),
        in_specs=[
            pl.BlockSpec((R, HH), lambda i: (i, 0)),
            pl.BlockSpec((R, HH), lambda i: (i + NBLK_HI, 0)),
            pl.BlockSpec((R, HH), lambda i: (i, 0)),
            pl.BlockSpec((R, HH), lambda i: (i + N // R, 0)),
            pl.BlockSpec((R, 1), lambda i: (i, 0)),
            pl.BlockSpec((1, H), lambda i: (0, 0)),
            pl.BlockSpec((H, H), lambda i: (0, 0)),
        ],
        out_specs=pl.BlockSpec((NC, R, HH), lambda i: (0, i, 0)),
        out_shape=jax.ShapeDtypeStruct((NC, N, HH), _f32),
    )(acc2d, acc2d, g2d, g2d, dis, b, w2)


def _tc_final(acc2d, g2d, dis, b, wfc, bfc):
    def body(al_ref, ah_ref, gl_ref, gh_ref, dis_ref, b_ref, w_ref,
             bfc_ref, o_ref):
        a64 = jnp.concatenate([al_ref[...], ah_ref[...]], axis=1)
        g64 = jnp.concatenate([gl_ref[...], gh_ref[...]], axis=1)
        d = dis_ref[...]
        h = jnp.maximum(d * (a64 + g64) + b_ref[...], 0.0)
        o_ref[...] = (jnp.dot(h, w_ref[...], preferred_element_type=_f32)
                      + bfc_ref[0, 0])

    return pl.pallas_call(
        body,
        grid=(N // R,),
        in_specs=[
            pl.BlockSpec((R, HH), lambda i: (i, 0)),
            pl.BlockSpec((R, HH), lambda i: (i + NBLK_HI, 0)),
            pl.BlockSpec((R, HH), lambda i: (i, 0)),
            pl.BlockSpec((R, HH), lambda i: (i + N // R, 0)),
            pl.BlockSpec((R, 1), lambda i: (i, 0)),
            pl.BlockSpec((1, H), lambda i: (0, 0)),
            pl.BlockSpec((H, 1), lambda i: (0, 0)),
            pl.BlockSpec((1, 1), lambda i: (0, 0)),
        ],
        out_specs=pl.BlockSpec((R, 1), lambda i: (i, 0)),
        out_shape=jax.ShapeDtypeStruct((N, 1), _f32),
    )(acc2d, acc2d, g2d, g2d, dis, b, wfc, bfc)


@jax.jit
def kernel(x, edge_index, emb, W1, b1, W2, b2, Wfc, bfc):
    col = edge_index[1]

    pad = jnp.zeros((2, EPAD - E), jnp.int32).at[1].set(N)
    edge_p = jnp.concatenate([edge_index.astype(jnp.int32), pad], axis=1)
    edge_st = jnp.stack(
        [edge_p, edge_p.at[0].add(N)], axis=0)

    t1 = _mm_small(emb, W1)
    hist2 = _sc_hist(col)
    xp = jnp.pad(x, (0, NPG - N))
    lin1p = _sc_gather(t1, xp)

    g1_3d, dis = _tc_scale(hist2[0].reshape(NPH, 1), hist2[1].reshape(NPH, 1),
                           lin1p)
    g1 = g1_3d.reshape(NC * N, HH)

    a1 = _sc_edge(g1, edge_st)
    g2_3d = _tc_combine(a1, g1, dis, b1.reshape(1, H), W2)
    g2 = g2_3d.reshape(NC * N, HH)

    a2 = _sc_edge(g2, edge_st)
    out = _tc_final(a2, g2, dis, b2.reshape(1, H), Wfc, bfc.reshape(1, 1))
    return out.reshape(N)


# speedup vs baseline: 20.6277x; 1.1319x over previous
import dataclasses
import functools

import jax
import jax.numpy as jnp
from jax import lax
from jax.experimental import pallas as pl
from jax.experimental.pallas import tpu as pltpu
from jax.experimental.pallas import tpu_sc as plsc

N = 50000
E = 800000
H = 64
HH = 32
NTAGS = 1000
NC = 2
NS = 16
L = 16

NPH = 51200
HSLICE = NPH // NS
EPT_H = E // (NC * NS)
NPG = 50176
GPT = NPG // (NC * NS)
CH = 112
KB = 2
GRP = KB * CH
NGRP = 225
EPT = NGRP * GRP
EPAD = NS * EPT
EALLOC = EPAD + GRP
NPAD = 50400
RPT = NPAD // NS
FCH = 126
NF = RPT // FCH
NBLK_HI = NPAD // 400

def _sc_params():
    cp = pltpu.CompilerParams()
    fields = pltpu.CompilerParams.__dataclass_fields__
    if "needs_layout_passes" in fields:
        cp = dataclasses.replace(cp, needs_layout_passes=False)
    if "use_tc_tiling_on_sc" in fields:
        cp = dataclasses.replace(cp, use_tc_tiling_on_sc=False)
    return cp


@functools.lru_cache(maxsize=1)
def _mesh():
    return plsc.VectorSubcoreMesh(core_axis_name="c", subcore_axis_name="s",
                                  num_cores=NC, num_subcores=NS)
_f32 = jnp.float32


def _zero16():
    return jnp.zeros((L,), _f32)


@jax.jit
def _sc_hist(col):
    @functools.partial(
        pl.kernel,
        out_type=jax.ShapeDtypeStruct((NC, NPH), _f32),
        mesh=_mesh(),
        scratch_types=[
            pltpu.VMEM((1600,), jnp.int32),
            pltpu.VMEM((NPH,), _f32),
            pltpu.VMEM((HSLICE,), _f32),
            pltpu.VMEM((HSLICE,), _f32),
            pltpu.VMEM_SHARED((NS, NPH), _f32),
        ],
        compiler_params=_sc_params(),
    )
    def k(col_hbm, out_hbm, colbuf, hist, accv, tmpv, stage):
        c = lax.axis_index("c")
        s = lax.axis_index("s")
        ones = jnp.ones((L,), _f32)

        @pl.loop(0, NPH, step=L)
        def _(i):
            hist[pl.ds(i, L)] = _zero16()

        e0 = (c * NS + s) * EPT_H

        @pl.loop(0, 15)
        def _(j):
            pltpu.sync_copy(col_hbm.at[pl.ds(e0 + j * 1600, 1600)], colbuf)

            @pl.loop(0, 1600, step=L)
            def _(i):
                plsc.addupdate_scatter(hist, [colbuf[pl.ds(i, L)]], ones)

        pltpu.sync_copy(col_hbm.at[pl.ds(e0 + 24000, 1000)],
                        colbuf.at[pl.ds(0, 1000)])

        @pl.loop(0, 992, step=L)
        def _(i):
            plsc.addupdate_scatter(hist, [colbuf[pl.ds(i, L)]], ones)

        tail_mask = lax.iota(jnp.int32, L) < 8
        plsc.addupdate_scatter(hist, [colbuf[pl.ds(992, L)]], ones,
                               mask=tail_mask)

        pltpu.sync_copy(hist, stage.at[s])
        plsc.subcore_barrier()
        base = s * HSLICE

        @pl.loop(0, HSLICE, step=L)
        def _(i):
            accv[pl.ds(i, L)] = _zero16()

        @pl.loop(0, NS)
        def _(t):
            pltpu.sync_copy(stage.at[t, pl.ds(base, HSLICE)], tmpv)

            @pl.loop(0, HSLICE, step=L)
            def _(i):
                accv[pl.ds(i, L)] = accv[pl.ds(i, L)] + tmpv[pl.ds(i, L)]

        pltpu.sync_copy(accv, out_hbm.at[c, pl.ds(base, HSLICE)])

    return k(col)


@jax.jit
def _sc_gather(table, idx):
    @functools.partial(
        pl.kernel,
        out_type=jax.ShapeDtypeStruct((NPG, H), _f32),
        mesh=_mesh(),
        scratch_types=[
            pltpu.VMEM((128,), jnp.int32),
            pltpu.VMEM((128, H), _f32),
            pltpu.VMEM((32,), jnp.int32),
            pltpu.VMEM((32, H), _f32),
            pltpu.SemaphoreType.DMA,
        ],
        compiler_params=_sc_params(),
    )
    def k(t_hbm, x_hbm, out_hbm, idxv, rows, idxt, rowst, sem):
        w = lax.axis_index("s") * NC + lax.axis_index("c")
        base = w * GPT

        @pl.loop(0, 12)
        def _(j):
            b = base + j * 128
            pltpu.sync_copy(x_hbm.at[pl.ds(b, 128)], idxv)
            pltpu.async_copy(t_hbm.at[idxv], rows, sem).wait()
            pltpu.sync_copy(rows, out_hbm.at[pl.ds(b, 128), :])

        b = base + 1536
        pltpu.sync_copy(x_hbm.at[pl.ds(b, 32)], idxt)
        pltpu.async_copy(t_hbm.at[idxt], rowst, sem).wait()
        pltpu.sync_copy(rowst, out_hbm.at[pl.ds(b, 32), :])

    return k(table, idx)


@jax.jit
def _sc_edge(g2d, edge_st):
    @functools.partial(
        pl.kernel,
        out_type=jax.ShapeDtypeStruct((NC * NPAD, HH), _f32),
        mesh=_mesh(),
        scratch_types=[
            pltpu.VMEM((3, KB, 2, CH), jnp.int32),
            pltpu.VMEM((3, KB, CH, HH), _f32),
            pltpu.VMEM((FCH, HH), _f32),
            pltpu.VMEM_SHARED((NPAD, HH), _f32),
        ] + [pltpu.SemaphoreType.DMA] * 10,
        compiler_params=_sc_params(),
    )
    def k(g_hbm, e_hbm, out_hbm, idxb, rows, zb, acc,
          sa0, sa1, sa2, sg0, sg1, sg2, ss0, ss1, ss2, sz):
        c = lax.axis_index("c")
        s = lax.axis_index("s")
        sa = [sa0, sa1, sa2]
        sg = [sg0, sg1, sg2]
        ss = [ss0, ss1, ss2]

        @pl.loop(0, FCH)
        def _(i):
            zb[i, pl.ds(0, L)] = _zero16()
            zb[i, pl.ds(L, L)] = _zero16()

        r0 = s * RPT
        zd = [pltpu.async_copy(zb, acc.at[pl.ds(r0 + i * FCH, FCH), :],
                               sz) for i in range(NF)]
        for d in zd:
            d.wait()
        plsc.subcore_barrier()

        e0 = s * EPT

        def fire_a(g, p):
            for nb in range(KB):
                pltpu.async_copy(
                    e_hbm.at[c, :, pl.ds(e0 + g * GRP + nb * CH, CH)],
                    idxb.at[p, nb], sa[p])

        def drain_a(p):
            for nb in range(KB):
                pltpu.make_async_copy(e_hbm.at[c, :, pl.ds(0, CH)],
                                      idxb.at[p, nb], sa[p]).wait()

        def fire_c(p):
            for nb in range(KB):
                pltpu.async_copy(g_hbm.at[idxb.at[p, nb, 0]],
                                 rows.at[p, nb], sg[p])

        def drain_c(p):
            for nb in range(KB):
                pltpu.make_async_copy(g_hbm.at[pl.ds(0, CH), :],
                                      rows.at[p, nb], sg[p]).wait()

        def fire_d(p):
            for nb in range(KB):
                pltpu.async_copy(rows.at[p, nb], acc.at[idxb.at[p, nb, 1]],
                                 ss[p], add=True)

        def drain_d(p):
            for nb in range(KB):
                pltpu.make_async_copy(g_hbm.at[pl.ds(0, CH), :],
                                      rows.at[p, nb], ss[p]).wait()

        fire_a(0, 0)
        drain_a(0); fire_c(0); fire_a(1, 1)
        drain_a(1); fire_c(1); drain_c(0); fire_d(0); fire_a(2, 2)
        drain_a(2); fire_c(2); drain_c(1); fire_d(1); drain_d(0); fire_a(3, 0)

        @pl.loop(1, NGRP // 3)
        def _(G):
            for p in range(3):
                g = 3 * G + p
                p1 = (p - 1) % 3
                p2 = (p - 2) % 3
                drain_a(p); fire_c(p)
                drain_c(p1); fire_d(p1)
                drain_d(p2); fire_a(g + 1, p2)

        drain_c(2); fire_d(2)
        drain_d(1)
        drain_d(2)
        drain_a(0)

        plsc.subcore_barrier()
        o0 = c * NPAD + r0
        fd = [pltpu.async_copy(acc.at[pl.ds(r0 + i * FCH, FCH), :],
                               out_hbm.at[pl.ds(o0 + i * FCH, FCH), :],
                               sz) for i in range(NF)]
        for d in fd:
            d.wait()

    return k(g2d, edge_st)


R = 400


def _mm_small(a, b):
    def body(a_ref, b_ref, o_ref):
        o_ref[...] = jnp.dot(a_ref[...], b_ref[...],
                             preferred_element_type=_f32)

    return pl.pallas_call(
        body,
        out_shape=jax.ShapeDtypeStruct((a.shape[0], b.shape[1]), _f32),
    )(a, b)


def _tc_scale(h0, h1, lin1p):
    def body(h0_ref, h1_ref, lin_ref, g_ref, dis_ref):
        deg = h0_ref[...] + h1_ref[...] + 1.0
        dis = lax.rsqrt(deg)
        g = lin_ref[...] * dis
        g_ref[...] = jnp.stack([g[:, :HH], g[:, HH:]], axis=0)
        dis_ref[...] = dis

    return pl.pallas_call(
        body,
        grid=(N // R,),
        in_specs=[
            pl.BlockSpec((R, 1), lambda i: (i, 0)),
            pl.BlockSpec((R, 1), lambda i: (i, 0)),
            pl.BlockSpec((R, H), lambda i: (i, 0)),
        ],
        out_specs=[
            pl.BlockSpec((NC, R, HH), lambda i: (0, i, 0)),
            pl.BlockSpec((R, 1), lambda i: (i, 0)),
        ],
        out_shape=[
            jax.ShapeDtypeStruct((NC, N, HH), _f32),
            jax.ShapeDtypeStruct((N, 1), _f32),
        ],
    )(h0, h1, lin1p)


def _tc_combine(acc2d, g2d, dis, b, w2):
    def body(al_ref, ah_ref, gl_ref, gh_ref, dis_ref, b_ref, w_ref, o_ref):
        a64 = jnp.concatenate([al_ref[...], ah_ref[...]], axis=1)
        g64 = jnp.concatenate([gl_ref[...], gh_ref[...]], axis=1)
        d = dis_ref[...]
        h = jnp.maximum(d * (a64 + g64) + b_ref[...], 0.0)
        lin = jnp.dot(h, w_ref[...], preferred_element_type=_f32)
        gn = lin * d
        o_ref[...] = jnp.stack([gn[:, :HH], gn[:, HH:]], axis=0)

    return pl.pallas_call(
        body,
        grid=(N // R,),
        in_specs=[
            pl.BlockSpec((R, HH), lambda i: (i, 0)),
            pl.BlockSpec((R, HH), lambda i: (i + NBLK_HI, 0)),
            pl.BlockSpec((R, HH), lambda i: (i, 0)),
            pl.BlockSpec((R, HH), lambda i: (i + N // R, 0)),
            pl.BlockSpec((R, 1), lambda i: (i, 0)),
            pl.BlockSpec((1, H), lambda i: (0, 0)),
            pl.BlockSpec((H, H), lambda i: (0, 0)),
        ],
        out_specs=pl.BlockSpec((NC, R, HH), lambda i: (0, i, 0)),
        out_shape=jax.ShapeDtypeStruct((NC, N, HH), _f32),
    )(acc2d, acc2d, g2d, g2d, dis, b, w2)


def _tc_final(acc2d, g2d, dis, b, wfc, bfc):
    def body(al_ref, ah_ref, gl_ref, gh_ref, dis_ref, b_ref, w_ref,
             bfc_ref, o_ref):
        a64 = jnp.concatenate([al_ref[...], ah_ref[...]], axis=1)
        g64 = jnp.concatenate([gl_ref[...], gh_ref[...]], axis=1)
        d = dis_ref[...]
        h = jnp.maximum(d * (a64 + g64) + b_ref[...], 0.0)
        o_ref[...] = (jnp.dot(h, w_ref[...], preferred_element_type=_f32)
                      + bfc_ref[0, 0])

    return pl.pallas_call(
        body,
        grid=(N // R,),
        in_specs=[
            pl.BlockSpec((R, HH), lambda i: (i, 0)),
            pl.BlockSpec((R, HH), lambda i: (i + NBLK_HI, 0)),
            pl.BlockSpec((R, HH), lambda i: (i, 0)),
            pl.BlockSpec((R, HH), lambda i: (i + N // R, 0)),
            pl.BlockSpec((R, 1), lambda i: (i, 0)),
            pl.BlockSpec((1, H), lambda i: (0, 0)),
            pl.BlockSpec((H, 1), lambda i: (0, 0)),
            pl.BlockSpec((1, 1), lambda i: (0, 0)),
        ],
        out_specs=pl.BlockSpec((R, 1), lambda i: (i, 0)),
        out_shape=jax.ShapeDtypeStruct((N, 1), _f32),
    )(acc2d, acc2d, g2d, g2d, dis, b, wfc, bfc)


@jax.jit
def kernel(x, edge_index, emb, W1, b1, W2, b2, Wfc, bfc):
    col = edge_index[1]

    pad = jnp.zeros((2, EALLOC - E), jnp.int32).at[1].set(N)
    edge_p = jnp.concatenate([edge_index.astype(jnp.int32), pad], axis=1)
    edge_st = jnp.stack(
        [edge_p, edge_p.at[0].add(N)], axis=0)

    t1 = _mm_small(emb, W1)
    hist2 = _sc_hist(col)
    xp = jnp.pad(x, (0, NPG - N))
    lin1p = _sc_gather(t1, xp)

    g1_3d, dis = _tc_scale(hist2[0].reshape(NPH, 1), hist2[1].reshape(NPH, 1),
                           lin1p)
    g1 = g1_3d.reshape(NC * N, HH)

    a1 = _sc_edge(g1, edge_st)
    g2_3d = _tc_combine(a1, g1, dis, b1.reshape(1, H), W2)
    g2 = g2_3d.reshape(NC * N, HH)

    a2 = _sc_edge(g2, edge_st)
    out = _tc_final(a2, g2, dis, b2.reshape(1, H), Wfc, bfc.reshape(1, 1))
    return out.reshape(N)


# speedup vs baseline: 23.5478x; 1.1416x over previous
import dataclasses
import functools

import jax
import jax.numpy as jnp
from jax import lax
from jax.experimental import pallas as pl
from jax.experimental.pallas import tpu as pltpu
from jax.experimental.pallas import tpu_sc as plsc

N = 50000
E = 800000
H = 64
HH = 32
NTAGS = 1000
NC = 2
NS = 16
L = 16

NPH = 51200
HSLICE = NPH // NS
EPT_H = E // (NC * NS)
NPG = 50176
GPT = NPG // (NC * NS)
CH = 112
KB = 2
GRP = KB * CH
NGRP = 225
EPT = NGRP * GRP
EPAD = NS * EPT
EALLOC = EPAD + GRP
NPAD = 50008
RPT = N // NS
FCH = 125
NF = RPT // FCH

def _sc_params():
    cp = pltpu.CompilerParams()
    fields = pltpu.CompilerParams.__dataclass_fields__
    if "needs_layout_passes" in fields:
        cp = dataclasses.replace(cp, needs_layout_passes=False)
    if "use_tc_tiling_on_sc" in fields:
        cp = dataclasses.replace(cp, use_tc_tiling_on_sc=False)
    return cp


@functools.lru_cache(maxsize=1)
def _mesh():
    return plsc.VectorSubcoreMesh(core_axis_name="c", subcore_axis_name="s",
                                  num_cores=NC, num_subcores=NS)
_f32 = jnp.float32


def _zero16():
    return jnp.zeros((L,), _f32)


@jax.jit
def _sc_hist(col):
    @functools.partial(
        pl.kernel,
        out_type=jax.ShapeDtypeStruct((NC, NPH), _f32),
        mesh=_mesh(),
        scratch_types=[
            pltpu.VMEM((1600,), jnp.int32),
            pltpu.VMEM((NPH,), _f32),
            pltpu.VMEM((HSLICE,), _f32),
            pltpu.VMEM((HSLICE,), _f32),
            pltpu.VMEM_SHARED((NS, NPH), _f32),
        ],
        compiler_params=_sc_params(),
    )
    def k(col_hbm, out_hbm, colbuf, hist, accv, tmpv, stage):
        c = lax.axis_index("c")
        s = lax.axis_index("s")
        ones = jnp.ones((L,), _f32)

        @pl.loop(0, NPH, step=L)
        def _(i):
            hist[pl.ds(i, L)] = _zero16()

        e0 = (c * NS + s) * EPT_H

        @pl.loop(0, 15)
        def _(j):
            pltpu.sync_copy(col_hbm.at[pl.ds(e0 + j * 1600, 1600)], colbuf)

            @pl.loop(0, 1600, step=L)
            def _(i):
                plsc.addupdate_scatter(hist, [colbuf[pl.ds(i, L)]], ones)

        pltpu.sync_copy(col_hbm.at[pl.ds(e0 + 24000, 1000)],
                        colbuf.at[pl.ds(0, 1000)])

        @pl.loop(0, 992, step=L)
        def _(i):
            plsc.addupdate_scatter(hist, [colbuf[pl.ds(i, L)]], ones)

        tail_mask = lax.iota(jnp.int32, L) < 8
        plsc.addupdate_scatter(hist, [colbuf[pl.ds(992, L)]], ones,
                               mask=tail_mask)

        pltpu.sync_copy(hist, stage.at[s])
        plsc.subcore_barrier()
        base = s * HSLICE

        @pl.loop(0, HSLICE, step=L)
        def _(i):
            accv[pl.ds(i, L)] = _zero16()

        @pl.loop(0, NS)
        def _(t):
            pltpu.sync_copy(stage.at[t, pl.ds(base, HSLICE)], tmpv)

            @pl.loop(0, HSLICE, step=L)
            def _(i):
                accv[pl.ds(i, L)] = accv[pl.ds(i, L)] + tmpv[pl.ds(i, L)]

        pltpu.sync_copy(accv, out_hbm.at[c, pl.ds(base, HSLICE)])

    return k(col)


@jax.jit
def _sc_gather(table, idx):
    @functools.partial(
        pl.kernel,
        out_type=jax.ShapeDtypeStruct((NPG, H), _f32),
        mesh=_mesh(),
        scratch_types=[
            pltpu.VMEM((128,), jnp.int32),
            pltpu.VMEM((128, H), _f32),
            pltpu.VMEM((32,), jnp.int32),
            pltpu.VMEM((32, H), _f32),
            pltpu.SemaphoreType.DMA,
        ],
        compiler_params=_sc_params(),
    )
    def k(t_hbm, x_hbm, out_hbm, idxv, rows, idxt, rowst, sem):
        w = lax.axis_index("s") * NC + lax.axis_index("c")
        base = w * GPT

        @pl.loop(0, 12)
        def _(j):
            b = base + j * 128
            pltpu.sync_copy(x_hbm.at[pl.ds(b, 128)], idxv)
            pltpu.async_copy(t_hbm.at[idxv], rows, sem).wait()
            pltpu.sync_copy(rows, out_hbm.at[pl.ds(b, 128), :])

        b = base + 1536
        pltpu.sync_copy(x_hbm.at[pl.ds(b, 32)], idxt)
        pltpu.async_copy(t_hbm.at[idxt], rowst, sem).wait()
        pltpu.sync_copy(rowst, out_hbm.at[pl.ds(b, 32), :])

    return k(table, idx)


@jax.jit
def _sc_edge(g2d, edge_st):
    @functools.partial(
        pl.kernel,
        out_type=jax.ShapeDtypeStruct((N, H), _f32),
        mesh=_mesh(),
        scratch_types=[
            pltpu.VMEM((3, KB, 2, CH), jnp.int32),
            pltpu.VMEM((3, KB, CH, HH), _f32),
            pltpu.VMEM((FCH, HH), _f32),
            pltpu.VMEM_SHARED((NPAD, HH), _f32),
        ] + [pltpu.SemaphoreType.DMA] * 10,
        compiler_params=_sc_params(),
    )
    def k(g_hbm, e_hbm, out_hbm, idxb, rows, zb, acc,
          sa0, sa1, sa2, sg0, sg1, sg2, ss0, ss1, ss2, sz):
        c = lax.axis_index("c")
        s = lax.axis_index("s")
        sa = [sa0, sa1, sa2]
        sg = [sg0, sg1, sg2]
        ss = [ss0, ss1, ss2]

        @pl.loop(0, FCH)
        def _(i):
            zb[i, pl.ds(0, L)] = _zero16()
            zb[i, pl.ds(L, L)] = _zero16()

        r0 = s * RPT
        zd = [pltpu.async_copy(zb, acc.at[pl.ds(r0 + i * FCH, FCH), :],
                               sz) for i in range(NF)]
        for d in zd:
            d.wait()
        plsc.subcore_barrier()

        e0 = s * EPT

        def fire_a(g, p):
            for nb in range(KB):
                pltpu.async_copy(
                    e_hbm.at[c, :, pl.ds(e0 + g * GRP + nb * CH, CH)],
                    idxb.at[p, nb], sa[p])

        def drain_a(p):
            for nb in range(KB):
                pltpu.make_async_copy(e_hbm.at[c, :, pl.ds(0, CH)],
                                      idxb.at[p, nb], sa[p]).wait()

        def fire_c(p):
            for nb in range(KB):
                pltpu.async_copy(g_hbm.at[idxb.at[p, nb, 0]],
                                 rows.at[p, nb], sg[p])

        def drain_c(p):
            for nb in range(KB):
                pltpu.make_async_copy(g_hbm.at[pl.ds(0, CH), :],
                                      rows.at[p, nb], sg[p]).wait()

        def fire_d(p):
            for nb in range(KB):
                pltpu.async_copy(rows.at[p, nb], acc.at[idxb.at[p, nb, 1]],
                                 ss[p], add=True)

        def drain_d(p):
            for nb in range(KB):
                pltpu.make_async_copy(g_hbm.at[pl.ds(0, CH), :],
                                      rows.at[p, nb], ss[p]).wait()

        fire_a(0, 0)
        drain_a(0); fire_c(0); fire_a(1, 1)
        drain_a(1); fire_c(1); drain_c(0); fire_d(0); fire_a(2, 2)
        drain_a(2); fire_c(2); drain_c(1); fire_d(1); drain_d(0); fire_a(3, 0)

        @pl.loop(1, NGRP // 3)
        def _(G):
            for p in range(3):
                g = 3 * G + p
                p1 = (p - 1) % 3
                p2 = (p - 2) % 3
                drain_a(p); fire_c(p)
                drain_c(p1); fire_d(p1)
                drain_d(p2); fire_a(g + 1, p2)

        drain_c(2); fire_d(2)
        drain_d(1)
        drain_d(2)
        drain_a(0)

        plsc.subcore_barrier()
        co = c * HH
        fd = [pltpu.async_copy(
                  acc.at[pl.ds(r0 + i * FCH, FCH), :],
                  out_hbm.at[pl.ds(r0 + i * FCH, FCH), pl.ds(co, HH)],
                  sz) for i in range(NF)]
        for d in fd:
            d.wait()

    return k(g2d, edge_st)


R = 1000


def _mm_small(a, b):
    def body(a_ref, b_ref, o_ref):
        o_ref[...] = jnp.dot(a_ref[...], b_ref[...],
                             preferred_element_type=_f32)

    return pl.pallas_call(
        body,
        out_shape=jax.ShapeDtypeStruct((a.shape[0], b.shape[1]), _f32),
    )(a, b)


def _tc_scale(h0, h1, lin1p):
    def body(h0_ref, h1_ref, lin_ref, g_ref, dis_ref):
        deg = h0_ref[...] + h1_ref[...] + 1.0
        dis = lax.rsqrt(deg)
        g_ref[...] = lin_ref[...] * dis
        dis_ref[...] = dis

    return pl.pallas_call(
        body,
        grid=(N // R,),
        in_specs=[
            pl.BlockSpec((R, 1), lambda i: (i, 0)),
            pl.BlockSpec((R, 1), lambda i: (i, 0)),
            pl.BlockSpec((R, H), lambda i: (i, 0)),
        ],
        out_specs=[
            pl.BlockSpec((R, H), lambda i: (i, 0)),
            pl.BlockSpec((R, 1), lambda i: (i, 0)),
        ],
        out_shape=[
            jax.ShapeDtypeStruct((N, H), _f32),
            jax.ShapeDtypeStruct((N, 1), _f32),
        ],
    )(h0, h1, lin1p)


def _tc_combine(acc, g, dis, b, w2):
    def body(a_ref, g_ref, dis_ref, b_ref, w_ref, o_ref):
        d = dis_ref[...]
        h = jnp.maximum(d * (a_ref[...] + g_ref[...]) + b_ref[...], 0.0)
        o_ref[...] = jnp.dot(h, w_ref[...], preferred_element_type=_f32) * d

    return pl.pallas_call(
        body,
        grid=(N // R,),
        in_specs=[
            pl.BlockSpec((R, H), lambda i: (i, 0)),
            pl.BlockSpec((R, H), lambda i: (i, 0)),
            pl.BlockSpec((R, 1), lambda i: (i, 0)),
            pl.BlockSpec((1, H), lambda i: (0, 0)),
            pl.BlockSpec((H, H), lambda i: (0, 0)),
        ],
        out_specs=pl.BlockSpec((R, H), lambda i: (i, 0)),
        out_shape=jax.ShapeDtypeStruct((N, H), _f32),
    )(acc, g, dis, b, w2)


def _tc_final(acc, g, dis, b, wfc, bfc):
    def body(a_ref, g_ref, dis_ref, b_ref, w_ref, bfc_ref, o_ref):
        d = dis_ref[...]
        h = jnp.maximum(d * (a_ref[...] + g_ref[...]) + b_ref[...], 0.0)
        o_ref[...] = (jnp.dot(h, w_ref[...], preferred_element_type=_f32)
                      + bfc_ref[0, 0])

    return pl.pallas_call(
        body,
        grid=(N // R,),
        in_specs=[
            pl.BlockSpec((R, H), lambda i: (i, 0)),
            pl.BlockSpec((R, H), lambda i: (i, 0)),
            pl.BlockSpec((R, 1), lambda i: (i, 0)),
            pl.BlockSpec((1, H), lambda i: (0, 0)),
            pl.BlockSpec((H, 1), lambda i: (0, 0)),
            pl.BlockSpec((1, 1), lambda i: (0, 0)),
        ],
        out_specs=pl.BlockSpec((R, 1), lambda i: (i, 0)),
        out_shape=jax.ShapeDtypeStruct((N, 1), _f32),
    )(acc, g, dis, b, wfc, bfc)


@jax.jit
def kernel(x, edge_index, emb, W1, b1, W2, b2, Wfc, bfc):
    col = edge_index[1]

    npad = EALLOC - E
    ga = jnp.concatenate([edge_index[0] * 2, jnp.zeros((npad,), jnp.int32)])
    cp = jnp.concatenate([edge_index[1], jnp.full((npad,), N, jnp.int32)])
    edge_st = jnp.stack([jnp.stack([ga, cp]),
                         jnp.stack([ga + 1, cp])])

    t1 = _mm_small(emb, W1)
    hist2 = _sc_hist(col)
    xp = jnp.pad(x, (0, NPG - N))
    lin1p = _sc_gather(t1, xp)

    g1, dis = _tc_scale(hist2[0].reshape(NPH, 1), hist2[1].reshape(NPH, 1),
                        lin1p)

    a1 = _sc_edge(g1.reshape(2 * N, HH), edge_st)
    g2 = _tc_combine(a1, g1, dis, b1.reshape(1, H), W2)

    a2 = _sc_edge(g2.reshape(2 * N, HH), edge_st)
    out = _tc_final(a2, g2, dis, b2.reshape(1, H), Wfc, bfc.reshape(1, 1))
    return out.reshape(N)


# speedup vs baseline: 41.4498x; 1.7602x over previous
import dataclasses
import functools

import jax
import jax.numpy as jnp
from jax import lax
from jax.experimental import pallas as pl
from jax.experimental.pallas import tpu as pltpu
from jax.experimental.pallas import tpu_sc as plsc

N = 50000
E = 800000
H = 64
HH = 32
NTAGS = 1000
NC = 2
NS = 16
L = 16

NPH = 51200
HSLICE = NPH // NS
EPT_H = E // (NC * NS)
NPG = 50176
GPT = NPG // (NC * NS)
CH = 96
PF = 6
NBR = 6
NCH = 522
NSG = NCH // PF
NCHA = NCH + PF
EPT = NCH * CH
NPAD = 50008
RPT = N // NS
FCH = 125
NF = RPT // FCH

def _sc_params():
    cp = pltpu.CompilerParams()
    fields = pltpu.CompilerParams.__dataclass_fields__
    if "needs_layout_passes" in fields:
        cp = dataclasses.replace(cp, needs_layout_passes=False)
    if "use_tc_tiling_on_sc" in fields:
        cp = dataclasses.replace(cp, use_tc_tiling_on_sc=False)
    return cp


@functools.lru_cache(maxsize=1)
def _mesh():
    return plsc.VectorSubcoreMesh(core_axis_name="c", subcore_axis_name="s",
                                  num_cores=NC, num_subcores=NS)
_f32 = jnp.float32


def _zero16():
    return jnp.zeros((L,), _f32)


@jax.jit
def _sc_hist(col):
    @functools.partial(
        pl.kernel,
        out_type=jax.ShapeDtypeStruct((NC, NPH), _f32),
        mesh=_mesh(),
        scratch_types=[
            pltpu.VMEM((1600,), jnp.int32),
            pltpu.VMEM((NPH,), _f32),
            pltpu.VMEM((HSLICE,), _f32),
            pltpu.VMEM((HSLICE,), _f32),
            pltpu.VMEM_SHARED((NS, NPH), _f32),
        ],
        compiler_params=_sc_params(),
    )
    def k(col_hbm, out_hbm, colbuf, hist, accv, tmpv, stage):
        c = lax.axis_index("c")
        s = lax.axis_index("s")
        ones = jnp.ones((L,), _f32)

        @pl.loop(0, NPH, step=L)
        def _(i):
            hist[pl.ds(i, L)] = _zero16()

        e0 = (c * NS + s) * EPT_H

        @pl.loop(0, 15)
        def _(j):
            pltpu.sync_copy(col_hbm.at[pl.ds(e0 + j * 1600, 1600)], colbuf)

            @pl.loop(0, 1600, step=L)
            def _(i):
                plsc.addupdate_scatter(hist, [colbuf[pl.ds(i, L)]], ones)

        pltpu.sync_copy(col_hbm.at[pl.ds(e0 + 24000, 1000)],
                        colbuf.at[pl.ds(0, 1000)])

        @pl.loop(0, 992, step=L)
        def _(i):
            plsc.addupdate_scatter(hist, [colbuf[pl.ds(i, L)]], ones)

        tail_mask = lax.iota(jnp.int32, L) < 8
        plsc.addupdate_scatter(hist, [colbuf[pl.ds(992, L)]], ones,
                               mask=tail_mask)

        pltpu.sync_copy(hist, stage.at[s])
        plsc.subcore_barrier()
        base = s * HSLICE

        @pl.loop(0, HSLICE, step=L)
        def _(i):
            accv[pl.ds(i, L)] = _zero16()

        @pl.loop(0, NS)
        def _(t):
            pltpu.sync_copy(stage.at[t, pl.ds(base, HSLICE)], tmpv)

            @pl.loop(0, HSLICE, step=L)
            def _(i):
                accv[pl.ds(i, L)] = accv[pl.ds(i, L)] + tmpv[pl.ds(i, L)]

        pltpu.sync_copy(accv, out_hbm.at[c, pl.ds(base, HSLICE)])

    return k(col)


@jax.jit
def _sc_gather(table, idx):
    @functools.partial(
        pl.kernel,
        out_type=jax.ShapeDtypeStruct((NPG, H), _f32),
        mesh=_mesh(),
        scratch_types=[
            pltpu.VMEM((128,), jnp.int32),
            pltpu.VMEM((128, H), _f32),
            pltpu.VMEM((32,), jnp.int32),
            pltpu.VMEM((32, H), _f32),
            pltpu.SemaphoreType.DMA,
        ],
        compiler_params=_sc_params(),
    )
    def k(t_hbm, x_hbm, out_hbm, idxv, rows, idxt, rowst, sem):
        w = lax.axis_index("s") * NC + lax.axis_index("c")
        base = w * GPT

        @pl.loop(0, 12)
        def _(j):
            b = base + j * 128
            pltpu.sync_copy(x_hbm.at[pl.ds(b, 128)], idxv)
            pltpu.async_copy(t_hbm.at[idxv], rows, sem).wait()
            pltpu.sync_copy(rows, out_hbm.at[pl.ds(b, 128), :])

        b = base + 1536
        pltpu.sync_copy(x_hbm.at[pl.ds(b, 32)], idxt)
        pltpu.async_copy(t_hbm.at[idxt], rowst, sem).wait()
        pltpu.sync_copy(rowst, out_hbm.at[pl.ds(b, 32), :])

    return k(table, idx)


@jax.jit
def _sc_edge(g2d, edge_ch):
    @functools.partial(
        pl.kernel,
        out_type=jax.ShapeDtypeStruct((N, H), _f32),
        mesh=_mesh(),
        scratch_types=[
            pltpu.VMEM((3, PF, 2, CH), jnp.int32),
            pltpu.VMEM((NBR, CH, HH), _f32),
            pltpu.VMEM((FCH, HH), _f32),
            pltpu.VMEM_SHARED((NPAD, HH), _f32),
        ] + [pltpu.SemaphoreType.DMA] * (3 + 2 * NBR + 1),
        compiler_params=_sc_params(),
    )
    def k(g_hbm, e_hbm, out_hbm, ib, rows, zb, acc, *sems):
        c = lax.axis_index("c")
        s = lax.axis_index("s")
        sa = sems[0:3]
        sc_ = sems[3:3 + NBR]
        sd = sems[3 + NBR:3 + 2 * NBR]
        sz = sems[3 + 2 * NBR]

        @pl.loop(0, FCH)
        def _(i):
            zb[i, pl.ds(0, L)] = _zero16()
            zb[i, pl.ds(L, L)] = _zero16()

        r0 = s * RPT
        zd = [pltpu.async_copy(zb, acc.at[pl.ds(r0 + i * FCH, FCH), :],
                               sz) for i in range(NF)]
        for d in zd:
            d.wait()
        plsc.subcore_barrier()

        gsrc = g_hbm.at[pl.ds(c, 2 * N), :]

        def fire_a(sgi, b):
            pltpu.async_copy(e_hbm.at[s, pl.ds(sgi * PF, PF)],
                             ib.at[b], sa[b])

        def drain_a(b):
            pltpu.make_async_copy(e_hbm.at[s, pl.ds(0, PF)],
                                  ib.at[b], sa[b]).wait()

        def fire_c(kk, ibb):
            pltpu.async_copy(gsrc.at[ib.at[ibb, kk, 0]], rows.at[kk],
                             sc_[kk])

        def drain_c(kk):
            pltpu.make_async_copy(g_hbm.at[pl.ds(0, CH), :], rows.at[kk],
                                  sc_[kk]).wait()

        def fire_d(kk, ibb):
            pltpu.async_copy(rows.at[kk], acc.at[ib.at[ibb, kk, 1]],
                             sd[kk], add=True)

        def drain_d(kk):
            pltpu.make_async_copy(g_hbm.at[pl.ds(0, CH), :], rows.at[kk],
                                  sd[kk]).wait()

        def chunk_ops(q, kk, warm):
            if warm >= 6:
                drain_d(kk)
            fire_c(kk, q)
            if warm >= 4:
                dk = kk - 4
                dib = q if dk >= 0 else (q - 1) % 3
                drain_c(dk % PF)
                fire_d(dk % PF, dib)

        fire_a(0, 0)
        fire_a(1, 1)
        fire_a(2, 2)
        drain_a(0)
        for kk in range(PF):
            chunk_ops(0, kk, kk)
        @pl.loop(0, (NSG - 3) // 3)
        def _(G):
            for qo in (1, 2, 3):
                sgi = 3 * G + qo
                q = qo % 3
                drain_a(q)
                for kk in range(PF):
                    chunk_ops(q, kk, 10)
                    if kk == 3:
                        fire_a(sgi + 2, (q + 2) % 3)
        q_t0 = (NSG - 2) % 3
        q_t1 = (NSG - 1) % 3
        drain_a(q_t0)
        for kk in range(PF):
            chunk_ops(q_t0, kk, 10)
            if kk == 3:
                fire_a(NSG, NSG % 3)
        drain_a(q_t1)
        for kk in range(PF):
            chunk_ops(q_t1, kk, 10)
        drain_d((NCH - 6) % PF)
        drain_d((NCH - 5) % PF)
        for jj in range(NCH - 4, NCH):
            kk = jj % PF
            drain_c(kk)
            fire_d(kk, q_t1)
        for jj in range(NCH - 4, NCH):
            drain_d(jj % PF)
        drain_a(NSG % 3)

        plsc.subcore_barrier()
        co = c * HH
        fd = [pltpu.async_copy(
                  acc.at[pl.ds(r0 + i * FCH, FCH), :],
                  out_hbm.at[pl.ds(r0 + i * FCH, FCH), pl.ds(co, HH)],
                  sz) for i in range(NF)]
        for d in fd:
            d.wait()

    return k(g2d, edge_ch)


R2 = 1000


def _mm_small(a, b):
    def body(a_ref, b_ref, o_ref):
        o_ref[...] = jnp.dot(a_ref[...], b_ref[...],
                             preferred_element_type=_f32)

    return pl.pallas_call(
        body,
        out_shape=jax.ShapeDtypeStruct((a.shape[0], b.shape[1]), _f32),
    )(a, b)


def _tc_scale(h0, h1, lin128):
    def body(h0_ref, h1_ref, lin_ref, g_ref, dis_ref):
        d2 = lax.rsqrt(h0_ref[...] + h1_ref[...] + 1.0)
        db = jnp.concatenate(
            [jnp.broadcast_to(d2[:, 0:1], (R2, H)),
             jnp.broadcast_to(d2[:, 1:2], (R2, H))], axis=1)
        g_ref[...] = lin_ref[...] * db
        dis_ref[...] = d2

    return pl.pallas_call(
        body,
        grid=(N // 2 // R2,),
        in_specs=[
            pl.BlockSpec((R2, 2), lambda i: (i, 0)),
            pl.BlockSpec((R2, 2), lambda i: (i, 0)),
            pl.BlockSpec((R2, 2 * H), lambda i: (i, 0)),
        ],
        out_specs=[
            pl.BlockSpec((R2, 2 * H), lambda i: (i, 0)),
            pl.BlockSpec((R2, 2), lambda i: (i, 0)),
        ],
        out_shape=[
            jax.ShapeDtypeStruct((N // 2 + 2, 2 * H), _f32),
            jax.ShapeDtypeStruct((N // 2, 2), _f32),
        ],
    )(h0, h1, lin128)


def _tc_combine(acc128, g128, dis, b128, w2bd):
    def body(a_ref, g_ref, dis_ref, b_ref, w_ref, o_ref):
        d2 = dis_ref[...]
        db = jnp.concatenate(
            [jnp.broadcast_to(d2[:, 0:1], (R2, H)),
             jnp.broadcast_to(d2[:, 1:2], (R2, H))], axis=1)
        h = jnp.maximum(db * (a_ref[...] + g_ref[...]) + b_ref[...], 0.0)
        o_ref[...] = jnp.dot(h, w_ref[...],
                             preferred_element_type=_f32) * db

    return pl.pallas_call(
        body,
        grid=(N // 2 // R2,),
        in_specs=[
            pl.BlockSpec((R2, 2 * H), lambda i: (i, 0)),
            pl.BlockSpec((R2, 2 * H), lambda i: (i, 0)),
            pl.BlockSpec((R2, 2), lambda i: (i, 0)),
            pl.BlockSpec((1, 2 * H), lambda i: (0, 0)),
            pl.BlockSpec((2 * H, 2 * H), lambda i: (0, 0)),
        ],
        out_specs=pl.BlockSpec((R2, 2 * H), lambda i: (i, 0)),
        out_shape=jax.ShapeDtypeStruct((N // 2 + 2, 2 * H), _f32),
    )(acc128, g128, dis, b128, w2bd)


def _tc_final(acc128, g128, dis, b128, wfcbd, bfc):
    def body(a_ref, g_ref, dis_ref, b_ref, w_ref, bfc_ref, o_ref):
        d2 = dis_ref[...]
        db = jnp.concatenate(
            [jnp.broadcast_to(d2[:, 0:1], (R2, H)),
             jnp.broadcast_to(d2[:, 1:2], (R2, H))], axis=1)
        h = jnp.maximum(db * (a_ref[...] + g_ref[...]) + b_ref[...], 0.0)
        o_ref[...] = (jnp.dot(h, w_ref[...], preferred_element_type=_f32)
                      + bfc_ref[0, 0])

    return pl.pallas_call(
        body,
        grid=(N // 2 // R2,),
        in_specs=[
            pl.BlockSpec((R2, 2 * H), lambda i: (i, 0)),
            pl.BlockSpec((R2, 2 * H), lambda i: (i, 0)),
            pl.BlockSpec((R2, 2), lambda i: (i, 0)),
            pl.BlockSpec((1, 2 * H), lambda i: (0, 0)),
            pl.BlockSpec((2 * H, 2), lambda i: (0, 0)),
            pl.BlockSpec((1, 1), lambda i: (0, 0)),
        ],
        out_specs=pl.BlockSpec((R2, 2), lambda i: (i, 0)),
        out_shape=jax.ShapeDtypeStruct((N // 2, 2), _f32),
    )(acc128, g128, dis, b128, wfcbd, bfc)


@jax.jit
def kernel(x, edge_index, emb, W1, b1, W2, b2, Wfc, bfc):
    col = edge_index[1]

    npad = NS * EPT - E
    ga = jnp.concatenate([edge_index[0] * 2, jnp.zeros((npad,), jnp.int32)])
    cp = jnp.concatenate([edge_index[1], jnp.full((npad,), N, jnp.int32)])
    base = jnp.concatenate([ga.reshape(NS, NCH, 1, CH),
                            cp.reshape(NS, NCH, 1, CH)], axis=2)
    edge_st = jnp.pad(base, ((0, 0), (0, PF), (0, 0), (0, 0)))

    t1 = _mm_small(emb, W1)
    hist2 = _sc_hist(col)
    xp = jnp.pad(x, (0, NPG - N))
    lin1p = _sc_gather(t1, xp)

    w2bd = (jnp.zeros((2 * H, 2 * H), _f32)
            .at[:H, :H].set(W2).at[H:, H:].set(W2))
    wfcbd = (jnp.zeros((2 * H, 2), _f32)
             .at[:H, 0].set(Wfc[:, 0]).at[H:, 1].set(Wfc[:, 0]))
    b1d = jnp.tile(b1, 2).reshape(1, 2 * H)
    b2d = jnp.tile(b2, 2).reshape(1, 2 * H)

    g1, dis = _tc_scale(hist2[0].reshape(NPH // 2, 2),
                        hist2[1].reshape(NPH // 2, 2),
                        lin1p.reshape(NPG // 2, 2 * H))

    a1 = _sc_edge(g1.reshape(2 * N + 8, HH), edge_st)
    g2 = _tc_combine(a1.reshape(N // 2, 2 * H), g1, dis, b1d, w2bd)

    a2 = _sc_edge(g2.reshape(2 * N + 8, HH), edge_st)
    out = _tc_final(a2.reshape(N // 2, 2 * H), g2, dis, b2d, wfcbd,
                    bfc.reshape(1, 1))
    return out.reshape(N)
